# baseline (speedup 1.0000x reference)
import jax
import jax.numpy as jnp
from jax import lax
from jax.experimental import pallas as pl
from jax.experimental.pallas import tpu as pltpu

K = 4


def _score_rank_body(x_ref, w_ref, out_ref):
    _, T, HW, C = x_ref.shape
    fsum = jnp.sum(x_ref[...], axis=2)
    feat = fsum * (1.0 / HW)
    fb = feat.astype(jnp.bfloat16).astype(jnp.float32)
    wb = w_ref[...].astype(jnp.bfloat16).astype(jnp.float32)
    s = jnp.sum(fb * wb[:, None, :], axis=2)
    sj = jnp.broadcast_to(s[:, None, :], (1, T, T))
    st = jnp.broadcast_to(s[:, :, None], (1, T, T))
    jj = lax.broadcasted_iota(jnp.int32, (1, T, T), 2)
    tt = lax.broadcasted_iota(jnp.int32, (1, T, T), 1)
    before = (sj > st) | ((sj == st) & (jj < tt))
    rank = jnp.sum(before.astype(jnp.int32), axis=2)
    rk = jnp.broadcast_to(rank[:, None, :], (1, T, T))
    inds = jnp.sum(jnp.where(rk == tt, jj, 0), axis=2)
    out_ref[...] = inds[:, None, :]


def _topk_body(inds_ref, x_ref, topk_ref):
    j = pl.program_id(1)
    topk_ref[0, :, j, :] = x_ref[0, 0, :, :]


def _back_dma_body(inds_ref, x_any, back_any, sem):
    b = pl.program_id(0)
    nk = back_any.shape[2]
    for jj in range(nk):
        t = inds_ref[b, K + jj]
        pltpu.make_async_copy(
            x_any.at[b, :, t], back_any.at[b, :, jj], sem
        ).start()
    for jj in range(nk):
        t = inds_ref[b, K + jj]
        pltpu.make_async_copy(
            x_any.at[b, :, t], back_any.at[b, :, jj], sem
        ).wait()


def kernel(x, x_cls, score_w, score_b):
    B, C, T, H, W = x.shape
    HW = H * W
    NK = T - K
    w2 = score_w.reshape(1, C)

    xq = jnp.transpose(x.reshape(B, C, T, HW), (0, 2, 3, 1))

    sorted_inds = pl.pallas_call(
        _score_rank_body,
        grid=(B,),
        in_specs=[
            pl.BlockSpec((1, T, HW, C), lambda b: (b, 0, 0, 0)),
            pl.BlockSpec((1, C), lambda b: (0, 0)),
        ],
        out_specs=pl.BlockSpec((1, 1, T), lambda b: (b, 0, 0)),
        out_shape=jax.ShapeDtypeStruct((B, 1, T), jnp.int32),
    )(xq, w2)
    inds2 = sorted_inds.reshape(B, T)

    topk4 = pl.pallas_call(
        _topk_body,
        grid_spec=pltpu.PrefetchScalarGridSpec(
            num_scalar_prefetch=1,
            grid=(B, K),
            in_specs=[
                pl.BlockSpec(
                    (1, 1, HW, C),
                    lambda b, j, inds: (b, inds[b, j], 0, 0),
                ),
            ],
            out_specs=pl.BlockSpec(
                (1, HW, K, C), lambda b, j, inds: (b, 0, 0, 0)
            ),
        ),
        out_shape=jax.ShapeDtypeStruct((B, HW, K, C), jnp.float32),
    )(inds2, xq)

    frames_back = pl.pallas_call(
        _back_dma_body,
        grid_spec=pltpu.PrefetchScalarGridSpec(
            num_scalar_prefetch=1,
            grid=(B,),
            in_specs=[pl.BlockSpec(memory_space=pltpu.MemorySpace.HBM)],
            out_specs=pl.BlockSpec(memory_space=pltpu.MemorySpace.HBM),
            scratch_shapes=[pltpu.SemaphoreType.DMA],
        ),
        out_shape=jax.ShapeDtypeStruct((B, C, NK, H, W), jnp.float32),
    )(inds2, x)

    frames_topk = topk4.reshape(B, H, W, K, C)
    return (frames_topk, frames_back)


# speedup vs baseline: 39.0772x; 39.0772x over previous
import jax
import jax.numpy as jnp
from jax import lax
from jax.experimental import pallas as pl
from jax.experimental.pallas import tpu as pltpu

K = 4


def _score_rank_body(x_ref, w_ref, out_ref):
    _, T, HW, C = x_ref.shape
    fsum = jnp.sum(x_ref[...], axis=2)
    feat = fsum * (1.0 / HW)
    fb = feat.astype(jnp.bfloat16).astype(jnp.float32)
    wb = w_ref[...].astype(jnp.bfloat16).astype(jnp.float32)
    s = jnp.sum(fb * wb[:, None, :], axis=2)
    sj = jnp.broadcast_to(s[:, None, :], (1, T, T))
    st = jnp.broadcast_to(s[:, :, None], (1, T, T))
    jj = lax.broadcasted_iota(jnp.int32, (1, T, T), 2)
    tt = lax.broadcasted_iota(jnp.int32, (1, T, T), 1)
    before = (sj > st) | ((sj == st) & (jj < tt))
    rank = jnp.sum(before.astype(jnp.int32), axis=2)
    rk = jnp.broadcast_to(rank[:, None, :], (1, T, T))
    inds = jnp.sum(jnp.where(rk == tt, jj, 0), axis=2)
    out_ref[...] = inds[:, None, :]


def _topk_body(inds_ref, x_ref, topk_ref):
    j = pl.program_id(1)
    topk_ref[0, :, j, :] = x_ref[0, 0, :, :]


def _back_body(inds_ref, x_ref, back_ref):
    back_ref[...] = x_ref[...]


def kernel(x, x_cls, score_w, score_b):
    B, C, T, H, W = x.shape
    HW = H * W
    NK = T - K
    w2 = score_w.reshape(1, C)

    xq = jnp.transpose(x.reshape(B, C, T, HW), (0, 2, 3, 1))

    sorted_inds = pl.pallas_call(
        _score_rank_body,
        grid=(B,),
        in_specs=[
            pl.BlockSpec((1, T, HW, C), lambda b: (b, 0, 0, 0)),
            pl.BlockSpec((1, C), lambda b: (0, 0)),
        ],
        out_specs=pl.BlockSpec((1, 1, T), lambda b: (b, 0, 0)),
        out_shape=jax.ShapeDtypeStruct((B, 1, T), jnp.int32),
    )(xq, w2)
    inds2 = sorted_inds.reshape(B, T)

    topk4 = pl.pallas_call(
        _topk_body,
        grid_spec=pltpu.PrefetchScalarGridSpec(
            num_scalar_prefetch=1,
            grid=(B, K),
            in_specs=[
                pl.BlockSpec(
                    (1, 1, HW, C),
                    lambda b, j, inds: (b, inds[b, j], 0, 0),
                ),
            ],
            out_specs=pl.BlockSpec(
                (1, HW, K, C), lambda b, j, inds: (b, 0, 0, 0)
            ),
        ),
        out_shape=jax.ShapeDtypeStruct((B, HW, K, C), jnp.float32),
    )(inds2, xq)

    back4 = pl.pallas_call(
        _back_body,
        grid_spec=pltpu.PrefetchScalarGridSpec(
            num_scalar_prefetch=1,
            grid=(B, NK),
            in_specs=[
                pl.BlockSpec(
                    (1, 1, HW, C),
                    lambda b, j, inds: (b, inds[b, j + K], 0, 0),
                ),
            ],
            out_specs=pl.BlockSpec(
                (1, 1, HW, C), lambda b, j, inds: (b, j, 0, 0)
            ),
        ),
        out_shape=jax.ShapeDtypeStruct((B, NK, HW, C), jnp.float32),
    )(inds2, xq)

    frames_topk = topk4.reshape(B, H, W, K, C)
    frames_back = jnp.transpose(back4, (0, 3, 1, 2)).reshape(B, C, NK, H, W)
    return (frames_topk, frames_back)
